# baseline (speedup 1.0000x reference)
import functools

import jax
import jax.numpy as jnp
from jax import lax
from jax.experimental import pallas as pl
from jax.experimental.pallas import tpu as pltpu
from jax.experimental.pallas import tpu_sc as plsc

_EMBED = 64
_ROW = 128
_L_OUT = 200
_NUM_WORKERS = 32


def _tc_format_table(table_t, lane_block=2048):
    v = table_t.shape[1]
    n_blocks = pl.cdiv(v, lane_block)

    def body(x_ref, o_ref):
        for k in range(lane_block // _ROW):
            t = x_ref[:, k * _ROW:(k + 1) * _ROW].T
            rows = pl.ds(k * _EMBED, _EMBED)
            o_ref[rows, 0:_EMBED] = t[0:_EMBED]
            o_ref[rows, _EMBED:_ROW] = t[_EMBED:_ROW]

    return pl.pallas_call(
        body,
        grid=(n_blocks,),
        in_specs=[pl.BlockSpec((_EMBED, lane_block), lambda i: (0, i))],
        out_specs=pl.BlockSpec((lane_block // 2, _ROW), lambda i: (i, 0)),
        out_shape=jax.ShapeDtypeStruct((n_blocks * lane_block // 2, _ROW),
                                       table_t.dtype),
    )(table_t)


def _sc_gather_rows(table2, flat_idx, window):
    n = flat_idx.shape[0]
    nw = n // window
    wpw = nw // _NUM_WORKERS
    assert nw * window == n and wpw * _NUM_WORKERS == nw and wpw % 2 == 0
    idx2 = flat_idx.reshape(_NUM_WORKERS, wpw, window)
    mesh = plsc.VectorSubcoreMesh(core_axis_name="core", subcore_axis_name="subcore")

    @functools.partial(
        pl.kernel,
        out_type=jax.ShapeDtypeStruct((n, _ROW), table2.dtype),
        mesh=mesh,
        scratch_types=[
            pltpu.VMEM((wpw, window), jnp.int32),
            pltpu.VMEM((window, _ROW), jnp.float32),
            pltpu.VMEM((window, _ROW), jnp.float32),
            pltpu.SemaphoreType.DMA,
            pltpu.SemaphoreType.DMA,
            pltpu.SemaphoreType.DMA,
            pltpu.SemaphoreType.DMA,
        ],
    )
    def gather_kernel(tab_hbm, idx_hbm, out_hbm, idx_v, buf_a, buf_b, gsem_a,
                      gsem_b, wsem_a, wsem_b):
        wid = lax.axis_index("subcore") * 2 + lax.axis_index("core")
        base = wid * wpw

        pltpu.sync_copy(idx_hbm.at[wid], idx_v)

        @pl.loop(0, wpw)
        def _(w):
            for j in range(window // 16):
                sl = pl.ds(j * 16, 16)
                v = idx_v[w, sl]
                idx_v[w, sl] = (
                    lax.shift_left(lax.shift_right_logical(v, 7), 6)
                    | (v & 63))

        def g_start(w, buf, sem):
            pltpu.async_copy(tab_hbm.at[idx_v.at[w]], buf, sem)

        def g_wait(w, buf, sem):
            pltpu.make_async_copy(tab_hbm.at[idx_v.at[w]], buf, sem).wait()

        def out_at(w):
            off = pl.multiple_of((base + w) * window, window)
            return out_hbm.at[pl.ds(off, window)]

        def wb_start(w, buf, sem):
            pltpu.async_copy(buf, out_at(w), sem)

        def wb_wait(w, buf, sem):
            pltpu.make_async_copy(buf, out_at(w), sem).wait()

        npairs = wpw // 2
        g_start(0, buf_a, gsem_a)

        @pl.loop(0, npairs)
        def _(i):
            w0 = 2 * i
            g_start(w0 + 1, buf_b, gsem_b)
            g_wait(w0, buf_a, gsem_a)
            wb_start(w0, buf_a, wsem_a)
            g_wait(w0 + 1, buf_b, gsem_b)
            wb_start(w0 + 1, buf_b, wsem_b)

            @pl.when(i < npairs - 1)
            def _():
                wb_wait(w0, buf_a, wsem_a)
                g_start(w0 + 2, buf_a, gsem_a)
                wb_wait(w0 + 1, buf_b, wsem_b)

        wb_wait(wpw - 2, buf_a, wsem_a)
        wb_wait(wpw - 1, buf_b, wsem_b)

    return gather_kernel(table2, idx2)


def _tc_transpose_t(rows3, idx_t, block_b=128, block_l=40):
    l_in, b = idx_t.shape

    def sel_t(x, it):
        hi = (it & 64)[None, :, :]
        t = jnp.transpose(x, (2, 1, 0))
        return jnp.where(hi == 0, t[:_EMBED], t[_EMBED:])

    if l_in == _L_OUT:
        def body(x_ref, i_ref, o_ref):
            o_ref[...] = sel_t(x_ref[...], i_ref[...])

        return pl.pallas_call(
            body,
            grid=(b // block_b, l_in // block_l),
            in_specs=[
                pl.BlockSpec((block_b, block_l, _ROW), lambda i, j: (i, j, 0)),
                pl.BlockSpec((block_l, block_b), lambda i, j: (j, i)),
            ],
            out_specs=pl.BlockSpec((_EMBED, block_l, block_b),
                                   lambda i, j: (0, j, i)),
            out_shape=jax.ShapeDtypeStruct((_EMBED, _L_OUT, b), rows3.dtype),
        )(rows3, idx_t)

    def body(x_ref, i_ref, o_ref):
        t = sel_t(x_ref[...], i_ref[...])
        pad = jnp.zeros((_EMBED, _L_OUT - l_in, block_b), t.dtype)
        o_ref[...] = jnp.concatenate([t, pad], axis=1)

    return pl.pallas_call(
        body,
        grid=(b // block_b,),
        in_specs=[
            pl.BlockSpec((block_b, l_in, _ROW), lambda i: (i, 0, 0)),
            pl.BlockSpec((l_in, block_b), lambda i: (0, i)),
        ],
        out_specs=pl.BlockSpec((_EMBED, _L_OUT, block_b), lambda i: (0, 0, i)),
        out_shape=jax.ShapeDtypeStruct((_EMBED, _L_OUT, b), rows3.dtype),
    )(rows3, idx_t)


def kernel(query_input, document_input, embedding_table):
    bq, lq = query_input.shape
    bd, ld = document_input.shape
    table_t = jnp.swapaxes(embedding_table, 0, 1)
    table2 = _tc_format_table(table_t)

    q_rows = _sc_gather_rows(table2, query_input.reshape(-1), window=64)
    d_rows = _sc_gather_rows(table2, document_input.reshape(-1), window=128)

    q_t = _tc_transpose_t(q_rows.reshape(bq, lq, _ROW),
                          jnp.swapaxes(query_input, 0, 1))
    d_t = _tc_transpose_t(d_rows.reshape(bd, ld, _ROW),
                          jnp.swapaxes(document_input, 0, 1))
    return jnp.transpose(q_t, (2, 0, 1)), jnp.transpose(d_t, (2, 0, 1))


# speedup vs baseline: 1.1045x; 1.1045x over previous
import functools

import jax
import jax.numpy as jnp
from jax import lax
from jax.experimental import pallas as pl
from jax.experimental.pallas import tpu as pltpu
from jax.experimental.pallas import tpu_sc as plsc

_EMBED = 64
_ROW = 128
_L_OUT = 200
_NUM_WORKERS = 32


def _tc_format_table(table_t, lane_block=2048, chunk=512):
    v = table_t.shape[1]

    def body(x_ref, o_ref):
        for k in range(lane_block // chunk):
            rows = pl.ds(k * chunk, chunk)
            o_ref[rows, 0:_EMBED] = x_ref[:, k * chunk:(k + 1) * chunk].T

    return pl.pallas_call(
        body,
        grid=(pl.cdiv(v, lane_block),),
        in_specs=[pl.BlockSpec((_EMBED, lane_block), lambda i: (0, i))],
        out_specs=pl.BlockSpec((lane_block, _ROW), lambda i: (i, 0)),
        out_shape=jax.ShapeDtypeStruct((v, _ROW), table_t.dtype),
        compiler_params=pltpu.CompilerParams(
            dimension_semantics=("parallel",)),
    )(table_t)


def _sc_gather_rows(table2, flat_idx, window):
    n = flat_idx.shape[0]
    nw = n // window
    wpw = nw // _NUM_WORKERS
    assert nw * window == n and wpw * _NUM_WORKERS == nw and wpw % 2 == 0
    idx2 = flat_idx.reshape(_NUM_WORKERS, wpw, window)
    mesh = plsc.VectorSubcoreMesh(core_axis_name="core", subcore_axis_name="subcore")

    @functools.partial(
        pl.kernel,
        out_type=jax.ShapeDtypeStruct((n, _ROW), table2.dtype),
        mesh=mesh,
        scratch_types=[
            pltpu.VMEM((wpw, window), jnp.int32),
            pltpu.VMEM((window, _ROW), jnp.float32),
            pltpu.VMEM((window, _ROW), jnp.float32),
            pltpu.SemaphoreType.DMA,
            pltpu.SemaphoreType.DMA,
            pltpu.SemaphoreType.DMA,
            pltpu.SemaphoreType.DMA,
        ],
    )
    def gather_kernel(tab_hbm, idx_hbm, out_hbm, idx_v, buf_a, buf_b, gsem_a,
                      gsem_b, wsem_a, wsem_b):
        wid = lax.axis_index("subcore") * 2 + lax.axis_index("core")
        base = wid * wpw

        pltpu.sync_copy(idx_hbm.at[wid], idx_v)

        def g_start(w, buf, sem):
            pltpu.async_copy(tab_hbm.at[idx_v.at[w]], buf, sem)

        def g_wait(w, buf, sem):
            pltpu.make_async_copy(tab_hbm.at[idx_v.at[w]], buf, sem).wait()

        def out_at(w):
            off = pl.multiple_of((base + w) * window, window)
            return out_hbm.at[pl.ds(off, window)]

        def wb_start(w, buf, sem):
            pltpu.async_copy(buf, out_at(w), sem)

        def wb_wait(w, buf, sem):
            pltpu.make_async_copy(buf, out_at(w), sem).wait()

        npairs = wpw // 2
        g_start(0, buf_a, gsem_a)

        @pl.loop(0, npairs)
        def _(i):
            w0 = 2 * i
            g_start(w0 + 1, buf_b, gsem_b)
            g_wait(w0, buf_a, gsem_a)
            wb_start(w0, buf_a, wsem_a)
            g_wait(w0 + 1, buf_b, gsem_b)
            wb_start(w0 + 1, buf_b, wsem_b)

            @pl.when(i < npairs - 1)
            def _():
                wb_wait(w0, buf_a, wsem_a)
                g_start(w0 + 2, buf_a, gsem_a)
                wb_wait(w0 + 1, buf_b, wsem_b)

        wb_wait(wpw - 2, buf_a, wsem_a)
        wb_wait(wpw - 1, buf_b, wsem_b)

    return gather_kernel(table2, idx2)


def _tc_transpose_t(rows3, block_b=128, block_l=40):
    b, l_in, _ = rows3.shape

    if l_in == _L_OUT:
        def body(x_ref, o_ref):
            t = jnp.transpose(x_ref[...], (2, 1, 0))
            o_ref[...] = t[:_EMBED]

        return pl.pallas_call(
            body,
            grid=(b // block_b, l_in // block_l),
            in_specs=[
                pl.BlockSpec((block_b, block_l, _ROW), lambda i, j: (i, j, 0)),
            ],
            out_specs=pl.BlockSpec((_EMBED, block_l, block_b),
                                   lambda i, j: (0, j, i)),
            out_shape=jax.ShapeDtypeStruct((_EMBED, _L_OUT, b), rows3.dtype),
            compiler_params=pltpu.CompilerParams(
                dimension_semantics=("parallel", "parallel")),
        )(rows3)

    def body(x_ref, o_ref):
        t = jnp.transpose(x_ref[...], (2, 1, 0))[:_EMBED]
        pad = jnp.zeros((_EMBED, _L_OUT - l_in, block_b), t.dtype)
        o_ref[...] = jnp.concatenate([t, pad], axis=1)

    return pl.pallas_call(
        body,
        grid=(b // block_b,),
        in_specs=[
            pl.BlockSpec((block_b, l_in, _ROW), lambda i: (i, 0, 0)),
        ],
        out_specs=pl.BlockSpec((_EMBED, _L_OUT, block_b), lambda i: (0, 0, i)),
        out_shape=jax.ShapeDtypeStruct((_EMBED, _L_OUT, b), rows3.dtype),
        compiler_params=pltpu.CompilerParams(
            dimension_semantics=("parallel",)),
    )(rows3)


def kernel(query_input, document_input, embedding_table):
    bq, lq = query_input.shape
    bd, ld = document_input.shape
    table_t = jnp.swapaxes(embedding_table, 0, 1)
    table2 = _tc_format_table(table_t)

    q_rows = _sc_gather_rows(table2, query_input.reshape(-1), window=64)
    d_rows = _sc_gather_rows(table2, document_input.reshape(-1), window=128)

    q_t = _tc_transpose_t(q_rows.reshape(bq, lq, _ROW))
    d_t = _tc_transpose_t(d_rows.reshape(bd, ld, _ROW))
    return jnp.transpose(q_t, (2, 0, 1)), jnp.transpose(d_t, (2, 0, 1))


# speedup vs baseline: 1.5457x; 1.3995x over previous
import functools

import jax
import jax.numpy as jnp
from jax import lax
from jax.experimental import pallas as pl
from jax.experimental.pallas import tpu as pltpu
from jax.experimental.pallas import tpu_sc as plsc

_EMBED = 64
_ROW = 128
_L_OUT = 200
_NUM_WORKERS = 32


def _tc_format_table(table_t, lane_block=8192, chunk=512):
    v = table_t.shape[1]

    def body(x_ref, o_ref):
        for k in range(lane_block // chunk):
            rows = pl.ds(k * chunk, chunk)
            o_ref[rows, 0:_EMBED] = x_ref[:, k * chunk:(k + 1) * chunk].T

    return pl.pallas_call(
        body,
        grid=(pl.cdiv(v, lane_block),),
        in_specs=[pl.BlockSpec((_EMBED, lane_block), lambda i: (0, i))],
        out_specs=pl.BlockSpec((lane_block, _ROW), lambda i: (i, 0)),
        out_shape=jax.ShapeDtypeStruct((v, _ROW), table_t.dtype),
        compiler_params=pltpu.CompilerParams(
            dimension_semantics=("parallel",)),
    )(table_t)


def _sc_gather_rows(table2, flat_idx, window):
    n = flat_idx.shape[0]
    nw = n // window
    wpw = nw // _NUM_WORKERS
    assert nw * window == n and wpw * _NUM_WORKERS == nw and wpw % 2 == 0
    idx2 = flat_idx.reshape(_NUM_WORKERS, wpw, window)
    mesh = plsc.VectorSubcoreMesh(core_axis_name="core", subcore_axis_name="subcore")

    @functools.partial(
        pl.kernel,
        out_type=jax.ShapeDtypeStruct((n, _ROW), table2.dtype),
        mesh=mesh,
        scratch_types=[
            pltpu.VMEM((wpw, window), jnp.int32),
            pltpu.VMEM((window, _ROW), jnp.float32),
            pltpu.VMEM((window, _ROW), jnp.float32),
            pltpu.SemaphoreType.DMA,
            pltpu.SemaphoreType.DMA,
            pltpu.SemaphoreType.DMA,
            pltpu.SemaphoreType.DMA,
        ],
    )
    def gather_kernel(tab_hbm, idx_hbm, out_hbm, idx_v, buf_a, buf_b, gsem_a,
                      gsem_b, wsem_a, wsem_b):
        wid = lax.axis_index("subcore") * 2 + lax.axis_index("core")
        base = wid * wpw

        pltpu.sync_copy(idx_hbm.at[wid], idx_v)

        def g_start(w, buf, sem):
            pltpu.async_copy(tab_hbm.at[idx_v.at[w]], buf, sem)

        def g_wait(w, buf, sem):
            pltpu.make_async_copy(tab_hbm.at[idx_v.at[w]], buf, sem).wait()

        def out_at(w):
            off = pl.multiple_of((base + w) * window, window)
            return out_hbm.at[pl.ds(off, window)]

        def wb_start(w, buf, sem):
            pltpu.async_copy(buf, out_at(w), sem)

        def wb_wait(w, buf, sem):
            pltpu.make_async_copy(buf, out_at(w), sem).wait()

        npairs = wpw // 2
        g_start(0, buf_a, gsem_a)

        @pl.loop(0, npairs)
        def _(i):
            w0 = 2 * i
            g_start(w0 + 1, buf_b, gsem_b)
            g_wait(w0, buf_a, gsem_a)
            wb_start(w0, buf_a, wsem_a)
            g_wait(w0 + 1, buf_b, gsem_b)
            wb_start(w0 + 1, buf_b, wsem_b)

            @pl.when(i < npairs - 1)
            def _():
                wb_wait(w0, buf_a, wsem_a)
                g_start(w0 + 2, buf_a, gsem_a)
                wb_wait(w0 + 1, buf_b, wsem_b)

        wb_wait(wpw - 2, buf_a, wsem_a)
        wb_wait(wpw - 1, buf_b, wsem_b)

    return gather_kernel(table2, idx2)


def _tc_transpose_t(rows3, block_b=128, block_l=40):
    b, l_in, _ = rows3.shape

    if l_in == _L_OUT:
        def body(x_ref, o_ref):
            t = jnp.transpose(x_ref[...], (2, 1, 0))
            o_ref[...] = t[:_EMBED]

        return pl.pallas_call(
            body,
            grid=(b // block_b, l_in // block_l),
            in_specs=[
                pl.BlockSpec((block_b, block_l, _ROW), lambda i, j: (i, j, 0)),
            ],
            out_specs=pl.BlockSpec((_EMBED, block_l, block_b),
                                   lambda i, j: (0, j, i)),
            out_shape=jax.ShapeDtypeStruct((_EMBED, _L_OUT, b), rows3.dtype),
            compiler_params=pltpu.CompilerParams(
                dimension_semantics=("parallel", "parallel")),
        )(rows3)

    def body(x_ref, o_ref):
        t = jnp.transpose(x_ref[...], (2, 1, 0))[:_EMBED]
        pad = jnp.zeros((_EMBED, _L_OUT - l_in, block_b), t.dtype)
        o_ref[...] = jnp.concatenate([t, pad], axis=1)

    return pl.pallas_call(
        body,
        grid=(b // block_b,),
        in_specs=[
            pl.BlockSpec((block_b, l_in, _ROW), lambda i: (i, 0, 0)),
        ],
        out_specs=pl.BlockSpec((_EMBED, _L_OUT, block_b), lambda i: (0, 0, i)),
        out_shape=jax.ShapeDtypeStruct((_EMBED, _L_OUT, b), rows3.dtype),
        compiler_params=pltpu.CompilerParams(
            dimension_semantics=("parallel",)),
    )(rows3)


def kernel(query_input, document_input, embedding_table):
    bq, lq = query_input.shape
    bd, ld = document_input.shape
    table_t = jnp.swapaxes(embedding_table, 0, 1)
    table2 = _tc_format_table(table_t)

    q_rows = _sc_gather_rows(table2, query_input.reshape(-1), window=64)
    d_rows = _sc_gather_rows(table2, document_input.reshape(-1), window=128)

    q_t = _tc_transpose_t(q_rows.reshape(bq, lq, _ROW))
    d_t = _tc_transpose_t(d_rows.reshape(bd, ld, _ROW))
    return jnp.transpose(q_t, (2, 0, 1)), jnp.transpose(d_t, (2, 0, 1))


# speedup vs baseline: 1.6199x; 1.0480x over previous
import functools

import jax
import jax.numpy as jnp
from jax import lax
from jax.experimental import pallas as pl
from jax.experimental.pallas import tpu as pltpu
from jax.experimental.pallas import tpu_sc as plsc

_EMBED = 64
_ROW = 128
_L_OUT = 200
_NUM_WORKERS = 32


def _tc_format_table(table_t, lane_block=16384, chunk=512):
    v = table_t.shape[1]

    def body(x_ref, o_ref):
        for k in range(lane_block // chunk):
            rows = pl.ds(k * chunk, chunk)
            o_ref[rows, 0:_EMBED] = x_ref[:, k * chunk:(k + 1) * chunk].T

    return pl.pallas_call(
        body,
        grid=(pl.cdiv(v, lane_block),),
        in_specs=[pl.BlockSpec((_EMBED, lane_block), lambda i: (0, i))],
        out_specs=pl.BlockSpec((lane_block, _ROW), lambda i: (i, 0)),
        out_shape=jax.ShapeDtypeStruct((v, _ROW), table_t.dtype),
        compiler_params=pltpu.CompilerParams(
            dimension_semantics=("parallel",)),
    )(table_t)


def _sc_gather_rows(table2, flat_idx, window):
    n = flat_idx.shape[0]
    nw = n // window
    wpw = nw // _NUM_WORKERS
    assert nw * window == n and wpw * _NUM_WORKERS == nw and wpw % 2 == 0
    idx2 = flat_idx.reshape(_NUM_WORKERS, wpw, window)
    mesh = plsc.VectorSubcoreMesh(core_axis_name="core", subcore_axis_name="subcore")

    @functools.partial(
        pl.kernel,
        out_type=jax.ShapeDtypeStruct((n, _ROW), table2.dtype),
        mesh=mesh,
        scratch_types=[
            pltpu.VMEM((wpw, window), jnp.int32),
            pltpu.VMEM((window, _ROW), jnp.float32),
            pltpu.VMEM((window, _ROW), jnp.float32),
            pltpu.SemaphoreType.DMA,
            pltpu.SemaphoreType.DMA,
            pltpu.SemaphoreType.DMA,
            pltpu.SemaphoreType.DMA,
        ],
    )
    def gather_kernel(tab_hbm, idx_hbm, out_hbm, idx_v, buf_a, buf_b, gsem_a,
                      gsem_b, wsem_a, wsem_b):
        wid = lax.axis_index("subcore") * 2 + lax.axis_index("core")
        base = wid * wpw

        pltpu.sync_copy(idx_hbm.at[wid], idx_v)

        def g_start(w, buf, sem):
            pltpu.async_copy(tab_hbm.at[idx_v.at[w]], buf, sem)

        def g_wait(w, buf, sem):
            pltpu.make_async_copy(tab_hbm.at[idx_v.at[w]], buf, sem).wait()

        def out_at(w):
            off = pl.multiple_of((base + w) * window, window)
            return out_hbm.at[pl.ds(off, window)]

        def wb_start(w, buf, sem):
            pltpu.async_copy(buf, out_at(w), sem)

        def wb_wait(w, buf, sem):
            pltpu.make_async_copy(buf, out_at(w), sem).wait()

        npairs = wpw // 2
        g_start(0, buf_a, gsem_a)

        @pl.loop(0, npairs)
        def _(i):
            w0 = 2 * i
            g_start(w0 + 1, buf_b, gsem_b)
            g_wait(w0, buf_a, gsem_a)
            wb_start(w0, buf_a, wsem_a)
            g_wait(w0 + 1, buf_b, gsem_b)
            wb_start(w0 + 1, buf_b, wsem_b)

            @pl.when(i < npairs - 1)
            def _():
                wb_wait(w0, buf_a, wsem_a)
                g_start(w0 + 2, buf_a, gsem_a)
                wb_wait(w0 + 1, buf_b, wsem_b)

        wb_wait(wpw - 2, buf_a, wsem_a)
        wb_wait(wpw - 1, buf_b, wsem_b)

    return gather_kernel(table2, idx2)


def _tc_transpose_t(rows3, block_b=256, block_l=40):
    b, l_in, _ = rows3.shape

    if l_in == _L_OUT:
        def body(x_ref, o_ref):
            t = jnp.transpose(x_ref[...], (2, 1, 0))
            o_ref[...] = t[:_EMBED]

        return pl.pallas_call(
            body,
            grid=(b // block_b, l_in // block_l),
            in_specs=[
                pl.BlockSpec((block_b, block_l, _ROW), lambda i, j: (i, j, 0)),
            ],
            out_specs=pl.BlockSpec((_EMBED, block_l, block_b),
                                   lambda i, j: (0, j, i)),
            out_shape=jax.ShapeDtypeStruct((_EMBED, _L_OUT, b), rows3.dtype),
            compiler_params=pltpu.CompilerParams(
                dimension_semantics=("parallel", "parallel")),
        )(rows3)

    def body(x_ref, o_ref):
        t = jnp.transpose(x_ref[...], (2, 1, 0))[:_EMBED]
        pad = jnp.zeros((_EMBED, _L_OUT - l_in, block_b), t.dtype)
        o_ref[...] = jnp.concatenate([t, pad], axis=1)

    return pl.pallas_call(
        body,
        grid=(b // block_b,),
        in_specs=[
            pl.BlockSpec((block_b, l_in, _ROW), lambda i: (i, 0, 0)),
        ],
        out_specs=pl.BlockSpec((_EMBED, _L_OUT, block_b), lambda i: (0, 0, i)),
        out_shape=jax.ShapeDtypeStruct((_EMBED, _L_OUT, b), rows3.dtype),
        compiler_params=pltpu.CompilerParams(
            dimension_semantics=("parallel",)),
    )(rows3)


def kernel(query_input, document_input, embedding_table):
    bq, lq = query_input.shape
    bd, ld = document_input.shape
    table_t = jnp.swapaxes(embedding_table, 0, 1)
    table2 = _tc_format_table(table_t)

    q_rows = _sc_gather_rows(table2, query_input.reshape(-1), window=64)
    d_rows = _sc_gather_rows(table2, document_input.reshape(-1), window=128)

    q_t = _tc_transpose_t(q_rows.reshape(bq, lq, _ROW))
    d_t = _tc_transpose_t(d_rows.reshape(bd, ld, _ROW))
    return jnp.transpose(q_t, (2, 0, 1)), jnp.transpose(d_t, (2, 0, 1))


# speedup vs baseline: 1.6406x; 1.0128x over previous
import functools

import jax
import jax.numpy as jnp
from jax import lax
from jax.experimental import pallas as pl
from jax.experimental.pallas import tpu as pltpu
from jax.experimental.pallas import tpu_sc as plsc

_EMBED = 64
_ROW = 128
_L_OUT = 200
_NUM_WORKERS = 32


def _tc_format_table(table_t, lane_block=32768, chunk=512):
    v = table_t.shape[1]

    def body(x_ref, o_ref):
        for k in range(lane_block // chunk):
            rows = pl.ds(k * chunk, chunk)
            o_ref[rows, 0:_EMBED] = x_ref[:, k * chunk:(k + 1) * chunk].T

    return pl.pallas_call(
        body,
        grid=(pl.cdiv(v, lane_block),),
        in_specs=[pl.BlockSpec((_EMBED, lane_block), lambda i: (0, i))],
        out_specs=pl.BlockSpec((lane_block, _ROW), lambda i: (i, 0)),
        out_shape=jax.ShapeDtypeStruct((v, _ROW), table_t.dtype),
        compiler_params=pltpu.CompilerParams(
            dimension_semantics=("parallel",)),
    )(table_t)


def _sc_gather_rows(table2, flat_idx, window):
    n = flat_idx.shape[0]
    nw = n // window
    wpw = nw // _NUM_WORKERS
    assert nw * window == n and wpw * _NUM_WORKERS == nw and wpw % 2 == 0
    idx2 = flat_idx.reshape(_NUM_WORKERS, wpw, window)
    mesh = plsc.VectorSubcoreMesh(core_axis_name="core", subcore_axis_name="subcore")

    @functools.partial(
        pl.kernel,
        out_type=jax.ShapeDtypeStruct((n, _ROW), table2.dtype),
        mesh=mesh,
        scratch_types=[
            pltpu.VMEM((wpw, window), jnp.int32),
            pltpu.VMEM((window, _ROW), jnp.float32),
            pltpu.VMEM((window, _ROW), jnp.float32),
            pltpu.SemaphoreType.DMA,
            pltpu.SemaphoreType.DMA,
            pltpu.SemaphoreType.DMA,
            pltpu.SemaphoreType.DMA,
        ],
    )
    def gather_kernel(tab_hbm, idx_hbm, out_hbm, idx_v, buf_a, buf_b, gsem_a,
                      gsem_b, wsem_a, wsem_b):
        wid = lax.axis_index("subcore") * 2 + lax.axis_index("core")
        base = wid * wpw

        pltpu.sync_copy(idx_hbm.at[wid], idx_v)

        def g_start(w, buf, sem):
            pltpu.async_copy(tab_hbm.at[idx_v.at[w]], buf, sem)

        def g_wait(w, buf, sem):
            pltpu.make_async_copy(tab_hbm.at[idx_v.at[w]], buf, sem).wait()

        def out_at(w):
            off = pl.multiple_of((base + w) * window, window)
            return out_hbm.at[pl.ds(off, window)]

        def wb_start(w, buf, sem):
            pltpu.async_copy(buf, out_at(w), sem)

        def wb_wait(w, buf, sem):
            pltpu.make_async_copy(buf, out_at(w), sem).wait()

        npairs = wpw // 2
        g_start(0, buf_a, gsem_a)

        @pl.loop(0, npairs)
        def _(i):
            w0 = 2 * i
            g_start(w0 + 1, buf_b, gsem_b)
            g_wait(w0, buf_a, gsem_a)
            wb_start(w0, buf_a, wsem_a)
            g_wait(w0 + 1, buf_b, gsem_b)
            wb_start(w0 + 1, buf_b, wsem_b)

            @pl.when(i < npairs - 1)
            def _():
                wb_wait(w0, buf_a, wsem_a)
                g_start(w0 + 2, buf_a, gsem_a)
                wb_wait(w0 + 1, buf_b, wsem_b)

        wb_wait(wpw - 2, buf_a, wsem_a)
        wb_wait(wpw - 1, buf_b, wsem_b)

    return gather_kernel(table2, idx2)


def _tc_transpose_t(rows3, block_b=256, block_l=40):
    b, l_in, _ = rows3.shape

    if l_in == _L_OUT:
        def body(x_ref, o_ref):
            t = jnp.transpose(x_ref[...], (2, 1, 0))
            o_ref[...] = t[:_EMBED]

        return pl.pallas_call(
            body,
            grid=(b // block_b, l_in // block_l),
            in_specs=[
                pl.BlockSpec((block_b, block_l, _ROW), lambda i, j: (i, j, 0)),
            ],
            out_specs=pl.BlockSpec((_EMBED, block_l, block_b),
                                   lambda i, j: (0, j, i)),
            out_shape=jax.ShapeDtypeStruct((_EMBED, _L_OUT, b), rows3.dtype),
            compiler_params=pltpu.CompilerParams(
                dimension_semantics=("parallel", "parallel")),
        )(rows3)

    def body(x_ref, o_ref):
        t = jnp.transpose(x_ref[...], (2, 1, 0))[:_EMBED]
        pad = jnp.zeros((_EMBED, _L_OUT - l_in, block_b), t.dtype)
        o_ref[...] = jnp.concatenate([t, pad], axis=1)

    return pl.pallas_call(
        body,
        grid=(b // block_b,),
        in_specs=[
            pl.BlockSpec((block_b, l_in, _ROW), lambda i: (i, 0, 0)),
        ],
        out_specs=pl.BlockSpec((_EMBED, _L_OUT, block_b), lambda i: (0, 0, i)),
        out_shape=jax.ShapeDtypeStruct((_EMBED, _L_OUT, b), rows3.dtype),
        compiler_params=pltpu.CompilerParams(
            dimension_semantics=("parallel",)),
    )(rows3)


def kernel(query_input, document_input, embedding_table):
    bq, lq = query_input.shape
    bd, ld = document_input.shape
    table_t = jnp.swapaxes(embedding_table, 0, 1)
    table2 = _tc_format_table(table_t)

    q_rows = _sc_gather_rows(table2, query_input.reshape(-1), window=64)
    d_rows = _sc_gather_rows(table2, document_input.reshape(-1), window=128)

    q_t = _tc_transpose_t(q_rows.reshape(bq, lq, _ROW))
    d_t = _tc_transpose_t(d_rows.reshape(bd, ld, _ROW))
    return jnp.transpose(q_t, (2, 0, 1)), jnp.transpose(d_t, (2, 0, 1))
